# baseline (speedup 1.0000x reference)
import jax
import jax.numpy as jnp
from jax.experimental import pallas as pl
from jax.experimental.pallas import tpu as pltpu

_EPS = 1e-5
_BB = 16


def _lrelu(h):
    return jnp.where(h >= 0, h, 0.2 * h)


def _bdot(a, s):
    return jax.lax.dot_general(
        a, s, (((2,), (1,)), ((0,), (0,))), preferred_element_type=jnp.float32
    )


def _first_kernel(x_ref, c_ref, adj_ref, wx_ref, wc_ref, b_ref,
                  z_ref, ps_ref, pq_ref):
    bb, n, fx = x_ref.shape
    fc = c_ref.shape[-1]
    fo = wx_ref.shape[-1]
    s = jnp.dot(x_ref[...].reshape(bb * n, fx), wx_ref[...],
                preferred_element_type=jnp.float32)
    s = s + jnp.dot(c_ref[...].reshape(bb * n, fc), wc_ref[...],
                    preferred_element_type=jnp.float32)
    z = _bdot(adj_ref[...], s.reshape(bb, n, fo)) + b_ref[...]
    z_ref[...] = z
    ps_ref[...] = jnp.sum(z, axis=(0, 2)).reshape(1, 1, n)
    pq_ref[...] = jnp.sum(z * z, axis=(0, 2)).reshape(1, 1, n)


def _mid_kernel(zp_ref, adj_ref, sc_ref, sh_ref, w_ref, b_ref,
                z_ref, ps_ref, pq_ref):
    bb, n, fi = zp_ref.shape
    fo = w_ref.shape[-1]
    h = _lrelu(zp_ref[...] * sc_ref[...][None] + sh_ref[...][None])
    s = jnp.dot(h.reshape(bb * n, fi), w_ref[...],
                preferred_element_type=jnp.float32)
    z = _bdot(adj_ref[...], s.reshape(bb, n, fo)) + b_ref[...]
    z_ref[...] = z
    ps_ref[...] = jnp.sum(z, axis=(0, 2)).reshape(1, 1, n)
    pq_ref[...] = jnp.sum(z * z, axis=(0, 2)).reshape(1, 1, n)


def _last_gcn_kernel(zp_ref, adj_ref, sc_ref, sh_ref, w_ref, b_ref,
                     z_ref, ps_ref, pq_ref):
    bb, n, fi = zp_ref.shape
    h = _lrelu(zp_ref[...] * sc_ref[...][None] + sh_ref[...][None])
    s = jnp.sum(h * w_ref[...].reshape(1, 1, fi), axis=2)
    z = jnp.sum(adj_ref[...] * s[:, None, :], axis=2) + b_ref[...]
    z_ref[...] = z
    ps_ref[...] = jnp.sum(z, axis=0).reshape(1, 1, n)
    pq_ref[...] = jnp.sum(z * z, axis=0).reshape(1, 1, n)


def _head_kernel(z_ref, sc_ref, sh_ref, w5_ref, b5_ref, o_ref):
    h = _lrelu(z_ref[...] * sc_ref[...] + sh_ref[...])
    o = jnp.sum(h * w5_ref[...], axis=1, keepdims=True) + b5_ref[...]
    o_ref[...] = jax.nn.sigmoid(o)


def _finalize(ps, pq, cnt, g, be):
    s = jnp.sum(ps, axis=0).reshape(-1)
    q = jnp.sum(pq, axis=0).reshape(-1)
    mean = s / cnt
    var = q / cnt - mean * mean
    inv = jax.lax.rsqrt(var + _EPS)
    scale = g * inv
    shift = be - mean * scale
    return scale, shift


def kernel(x, adj, c, W1, b1, W2, b2, W3, b3, W4, b4,
           g1, be1, g2, be2, g3, be3, g4, be4, W5, b5):
    B, N, FX = x.shape
    FC = c.shape[-1]
    nblk = B // _BB
    grid = (nblk,)
    params = pltpu.CompilerParams(dimension_semantics=("parallel",))

    def blk(*shape):
        nd = len(shape)
        return pl.BlockSpec(shape, lambda i: (i,) + (0,) * (nd - 1))

    def full(*shape):
        nd = len(shape)
        return pl.BlockSpec(shape, lambda i: (0,) * nd)

    stats_shape = jax.ShapeDtypeStruct((nblk, 1, N), jnp.float32)
    stats_spec = pl.BlockSpec((1, 1, N), lambda i: (i, 0, 0))

    f1, f2, f3 = W1.shape[1], W2.shape[1], W3.shape[1]

    z1, ps, pq = pl.pallas_call(
        _first_kernel,
        grid=grid,
        in_specs=[blk(_BB, N, FX), blk(_BB, N, FC), blk(_BB, N, N),
                  full(FX, f1), full(FC, f1), full(1, f1)],
        out_specs=[blk(_BB, N, f1), stats_spec, stats_spec],
        out_shape=[jax.ShapeDtypeStruct((B, N, f1), jnp.float32),
                   stats_shape, stats_shape],
        compiler_params=params,
    )(x, c, adj, W1[:FX], W1[FX:], b1.reshape(1, f1))
    sc1, sh1 = _finalize(ps, pq, B * f1, g1, be1)

    def mid_pass(zp, fi, fo, w, b, sc, sh):
        return pl.pallas_call(
            _mid_kernel,
            grid=grid,
            in_specs=[blk(_BB, N, fi), blk(_BB, N, N),
                      full(N, 1), full(N, 1), full(fi, fo), full(1, fo)],
            out_specs=[blk(_BB, N, fo), stats_spec, stats_spec],
            out_shape=[jax.ShapeDtypeStruct((B, N, fo), jnp.float32),
                       stats_shape, stats_shape],
            compiler_params=params,
        )(zp, adj, sc.reshape(N, 1), sh.reshape(N, 1), w, b.reshape(1, fo))

    z2, ps, pq = mid_pass(z1, f1, f2, W2, b2, sc1, sh1)
    sc2, sh2 = _finalize(ps, pq, B * f2, g2, be2)

    z3, ps, pq = mid_pass(z2, f2, f3, W3, b3, sc2, sh2)
    sc3, sh3 = _finalize(ps, pq, B * f3, g3, be3)

    z4, ps, pq = pl.pallas_call(
        _last_gcn_kernel,
        grid=grid,
        in_specs=[blk(_BB, N, f3), blk(_BB, N, N),
                  full(N, 1), full(N, 1), full(f3, 1), full(1, 1)],
        out_specs=[blk(_BB, N), stats_spec, stats_spec],
        out_shape=[jax.ShapeDtypeStruct((B, N), jnp.float32),
                   stats_shape, stats_shape],
        compiler_params=params,
    )(z3, adj, sc3.reshape(N, 1), sh3.reshape(N, 1), W4, b4.reshape(1, 1))
    sc4, sh4 = _finalize(ps, pq, B, g4, be4)

    out = pl.pallas_call(
        _head_kernel,
        out_shape=jax.ShapeDtypeStruct((B, 1), jnp.float32),
    )(z4, sc4.reshape(1, N), sh4.reshape(1, N), W5.reshape(1, N),
      b5.reshape(1, 1))
    return out


# speedup vs baseline: 1.3303x; 1.3303x over previous
import jax
import jax.numpy as jnp
from jax.experimental import pallas as pl
from jax.experimental.pallas import tpu as pltpu

_EPS = 1e-5
_BB = 64


def _lrelu(h):
    return jnp.where(h >= 0, h, 0.2 * h)


def _bdot(a, s):
    return jax.lax.dot_general(
        a, s, (((2,), (1,)), ((0,), (0,))), preferred_element_type=jnp.float32
    )


def _first_kernel(x_ref, c_ref, adj_ref, wx_ref, wc_ref, b_ref,
                  z_ref, ps_ref, pq_ref):
    bb, n, fx = x_ref.shape
    fc = c_ref.shape[-1]
    fo = wx_ref.shape[-1]
    s = jnp.dot(x_ref[...].reshape(bb * n, fx), wx_ref[...],
                preferred_element_type=jnp.float32)
    s = s + jnp.dot(c_ref[...].reshape(bb * n, fc), wc_ref[...],
                    preferred_element_type=jnp.float32)
    z = _bdot(adj_ref[...], s.reshape(bb, n, fo)) + b_ref[...]
    z_ref[...] = z
    ps_ref[...] = jnp.sum(z, axis=(0, 2)).reshape(1, 1, n)
    pq_ref[...] = jnp.sum(z * z, axis=(0, 2)).reshape(1, 1, n)


def _mid_kernel(zp_ref, adj_ref, sc_ref, sh_ref, w_ref, b_ref,
                z_ref, ps_ref, pq_ref):
    bb, n, fi = zp_ref.shape
    fo = w_ref.shape[-1]
    h = _lrelu(zp_ref[...] * sc_ref[...][None] + sh_ref[...][None])
    s = jnp.dot(h.reshape(bb * n, fi), w_ref[...],
                preferred_element_type=jnp.float32)
    z = _bdot(adj_ref[...], s.reshape(bb, n, fo)) + b_ref[...]
    z_ref[...] = z
    ps_ref[...] = jnp.sum(z, axis=(0, 2)).reshape(1, 1, n)
    pq_ref[...] = jnp.sum(z * z, axis=(0, 2)).reshape(1, 1, n)


def _last_gcn_kernel(zp_ref, adj_ref, sc_ref, sh_ref, w_ref, b_ref,
                     z_ref, ps_ref, pq_ref):
    bb, n, fi = zp_ref.shape
    h = _lrelu(zp_ref[...] * sc_ref[...][None] + sh_ref[...][None])
    s = jnp.sum(h * w_ref[...].reshape(1, 1, fi), axis=2)
    z = jnp.sum(adj_ref[...] * s[:, None, :], axis=2) + b_ref[...]
    z_ref[...] = z
    ps_ref[...] = jnp.sum(z, axis=0).reshape(1, 1, n)
    pq_ref[...] = jnp.sum(z * z, axis=0).reshape(1, 1, n)


def _head_kernel(z_ref, sc_ref, sh_ref, w5_ref, b5_ref, o_ref):
    h = _lrelu(z_ref[...] * sc_ref[...] + sh_ref[...])
    o = jnp.sum(h * w5_ref[...], axis=1, keepdims=True) + b5_ref[...]
    o_ref[...] = jax.nn.sigmoid(o)


def _finalize(ps, pq, cnt, g, be):
    s = jnp.sum(ps, axis=0).reshape(-1)
    q = jnp.sum(pq, axis=0).reshape(-1)
    mean = s / cnt
    var = q / cnt - mean * mean
    inv = jax.lax.rsqrt(var + _EPS)
    scale = g * inv
    shift = be - mean * scale
    return scale, shift


def kernel(x, adj, c, W1, b1, W2, b2, W3, b3, W4, b4,
           g1, be1, g2, be2, g3, be3, g4, be4, W5, b5):
    B, N, FX = x.shape
    FC = c.shape[-1]
    nblk = B // _BB
    grid = (nblk,)
    params = pltpu.CompilerParams(dimension_semantics=("parallel",))

    def blk(*shape):
        nd = len(shape)
        return pl.BlockSpec(shape, lambda i: (i,) + (0,) * (nd - 1))

    def full(*shape):
        nd = len(shape)
        return pl.BlockSpec(shape, lambda i: (0,) * nd)

    stats_shape = jax.ShapeDtypeStruct((nblk, 1, N), jnp.float32)
    stats_spec = pl.BlockSpec((1, 1, N), lambda i: (i, 0, 0))

    f1, f2, f3 = W1.shape[1], W2.shape[1], W3.shape[1]

    z1, ps, pq = pl.pallas_call(
        _first_kernel,
        grid=grid,
        in_specs=[blk(_BB, N, FX), blk(_BB, N, FC), blk(_BB, N, N),
                  full(FX, f1), full(FC, f1), full(1, f1)],
        out_specs=[blk(_BB, N, f1), stats_spec, stats_spec],
        out_shape=[jax.ShapeDtypeStruct((B, N, f1), jnp.float32),
                   stats_shape, stats_shape],
        compiler_params=params,
    )(x, c, adj, W1[:FX], W1[FX:], b1.reshape(1, f1))
    sc1, sh1 = _finalize(ps, pq, B * f1, g1, be1)

    def mid_pass(zp, fi, fo, w, b, sc, sh):
        return pl.pallas_call(
            _mid_kernel,
            grid=grid,
            in_specs=[blk(_BB, N, fi), blk(_BB, N, N),
                      full(N, 1), full(N, 1), full(fi, fo), full(1, fo)],
            out_specs=[blk(_BB, N, fo), stats_spec, stats_spec],
            out_shape=[jax.ShapeDtypeStruct((B, N, fo), jnp.float32),
                       stats_shape, stats_shape],
            compiler_params=params,
        )(zp, adj, sc.reshape(N, 1), sh.reshape(N, 1), w, b.reshape(1, fo))

    z2, ps, pq = mid_pass(z1, f1, f2, W2, b2, sc1, sh1)
    sc2, sh2 = _finalize(ps, pq, B * f2, g2, be2)

    z3, ps, pq = mid_pass(z2, f2, f3, W3, b3, sc2, sh2)
    sc3, sh3 = _finalize(ps, pq, B * f3, g3, be3)

    z4, ps, pq = pl.pallas_call(
        _last_gcn_kernel,
        grid=grid,
        in_specs=[blk(_BB, N, f3), blk(_BB, N, N),
                  full(N, 1), full(N, 1), full(f3, 1), full(1, 1)],
        out_specs=[blk(_BB, N), stats_spec, stats_spec],
        out_shape=[jax.ShapeDtypeStruct((B, N), jnp.float32),
                   stats_shape, stats_shape],
        compiler_params=params,
    )(z3, adj, sc3.reshape(N, 1), sh3.reshape(N, 1), W4, b4.reshape(1, 1))
    sc4, sh4 = _finalize(ps, pq, B, g4, be4)

    out = pl.pallas_call(
        _head_kernel,
        out_shape=jax.ShapeDtypeStruct((B, 1), jnp.float32),
    )(z4, sc4.reshape(1, N), sh4.reshape(1, N), W5.reshape(1, N),
      b5.reshape(1, 1))
    return out


# speedup vs baseline: 1.3541x; 1.0179x over previous
import jax
import jax.numpy as jnp
from jax.experimental import pallas as pl
from jax.experimental.pallas import tpu as pltpu

_EPS = 1e-5
_BB = 128


def _lrelu(h):
    return jnp.where(h >= 0, h, 0.2 * h)


def _bdot(a, s):
    return jax.lax.dot_general(
        a, s, (((2,), (1,)), ((0,), (0,))), preferred_element_type=jnp.float32
    )


def _first_kernel(x_ref, c_ref, adj_ref, wx_ref, wc_ref, b_ref,
                  z_ref, ps_ref, pq_ref):
    bb, n, fx = x_ref.shape
    fc = c_ref.shape[-1]
    fo = wx_ref.shape[-1]
    s = jnp.dot(x_ref[...].reshape(bb * n, fx), wx_ref[...],
                preferred_element_type=jnp.float32)
    s = s + jnp.dot(c_ref[...].reshape(bb * n, fc), wc_ref[...],
                    preferred_element_type=jnp.float32)
    z = _bdot(adj_ref[...], s.reshape(bb, n, fo)) + b_ref[...]
    z_ref[...] = z
    ps_ref[...] = jnp.sum(z, axis=(0, 2)).reshape(1, 1, n)
    pq_ref[...] = jnp.sum(z * z, axis=(0, 2)).reshape(1, 1, n)


def _mid_kernel(zp_ref, adj_ref, sc_ref, sh_ref, w_ref, b_ref,
                z_ref, ps_ref, pq_ref):
    bb, n, fi = zp_ref.shape
    fo = w_ref.shape[-1]
    h = _lrelu(zp_ref[...] * sc_ref[...][None] + sh_ref[...][None])
    s = jnp.dot(h.reshape(bb * n, fi), w_ref[...],
                preferred_element_type=jnp.float32)
    z = _bdot(adj_ref[...], s.reshape(bb, n, fo)) + b_ref[...]
    z_ref[...] = z
    ps_ref[...] = jnp.sum(z, axis=(0, 2)).reshape(1, 1, n)
    pq_ref[...] = jnp.sum(z * z, axis=(0, 2)).reshape(1, 1, n)


def _last_gcn_kernel(zp_ref, adj_ref, sc_ref, sh_ref, w_ref, b_ref,
                     z_ref, ps_ref, pq_ref):
    bb, n, fi = zp_ref.shape
    h = _lrelu(zp_ref[...] * sc_ref[...][None] + sh_ref[...][None])
    s = jnp.sum(h * w_ref[...].reshape(1, 1, fi), axis=2)
    z = jnp.sum(adj_ref[...] * s[:, None, :], axis=2) + b_ref[...]
    z_ref[...] = z
    ps_ref[...] = jnp.sum(z, axis=0).reshape(1, 1, n)
    pq_ref[...] = jnp.sum(z * z, axis=0).reshape(1, 1, n)


def _head_kernel(z_ref, sc_ref, sh_ref, w5_ref, b5_ref, o_ref):
    h = _lrelu(z_ref[...] * sc_ref[...] + sh_ref[...])
    o = jnp.sum(h * w5_ref[...], axis=1, keepdims=True) + b5_ref[...]
    o_ref[...] = jax.nn.sigmoid(o)


def _finalize(ps, pq, cnt, g, be):
    s = jnp.sum(ps, axis=0).reshape(-1)
    q = jnp.sum(pq, axis=0).reshape(-1)
    mean = s / cnt
    var = q / cnt - mean * mean
    inv = jax.lax.rsqrt(var + _EPS)
    scale = g * inv
    shift = be - mean * scale
    return scale, shift


def kernel(x, adj, c, W1, b1, W2, b2, W3, b3, W4, b4,
           g1, be1, g2, be2, g3, be3, g4, be4, W5, b5):
    B, N, FX = x.shape
    FC = c.shape[-1]
    nblk = B // _BB
    grid = (nblk,)
    params = pltpu.CompilerParams(dimension_semantics=("parallel",))

    def blk(*shape):
        nd = len(shape)
        return pl.BlockSpec(shape, lambda i: (i,) + (0,) * (nd - 1))

    def full(*shape):
        nd = len(shape)
        return pl.BlockSpec(shape, lambda i: (0,) * nd)

    stats_shape = jax.ShapeDtypeStruct((nblk, 1, N), jnp.float32)
    stats_spec = pl.BlockSpec((1, 1, N), lambda i: (i, 0, 0))

    f1, f2, f3 = W1.shape[1], W2.shape[1], W3.shape[1]

    z1, ps, pq = pl.pallas_call(
        _first_kernel,
        grid=grid,
        in_specs=[blk(_BB, N, FX), blk(_BB, N, FC), blk(_BB, N, N),
                  full(FX, f1), full(FC, f1), full(1, f1)],
        out_specs=[blk(_BB, N, f1), stats_spec, stats_spec],
        out_shape=[jax.ShapeDtypeStruct((B, N, f1), jnp.float32),
                   stats_shape, stats_shape],
        compiler_params=params,
    )(x, c, adj, W1[:FX], W1[FX:], b1.reshape(1, f1))
    sc1, sh1 = _finalize(ps, pq, B * f1, g1, be1)

    def mid_pass(zp, fi, fo, w, b, sc, sh):
        return pl.pallas_call(
            _mid_kernel,
            grid=grid,
            in_specs=[blk(_BB, N, fi), blk(_BB, N, N),
                      full(N, 1), full(N, 1), full(fi, fo), full(1, fo)],
            out_specs=[blk(_BB, N, fo), stats_spec, stats_spec],
            out_shape=[jax.ShapeDtypeStruct((B, N, fo), jnp.float32),
                       stats_shape, stats_shape],
            compiler_params=params,
        )(zp, adj, sc.reshape(N, 1), sh.reshape(N, 1), w, b.reshape(1, fo))

    z2, ps, pq = mid_pass(z1, f1, f2, W2, b2, sc1, sh1)
    sc2, sh2 = _finalize(ps, pq, B * f2, g2, be2)

    z3, ps, pq = mid_pass(z2, f2, f3, W3, b3, sc2, sh2)
    sc3, sh3 = _finalize(ps, pq, B * f3, g3, be3)

    z4, ps, pq = pl.pallas_call(
        _last_gcn_kernel,
        grid=grid,
        in_specs=[blk(_BB, N, f3), blk(_BB, N, N),
                  full(N, 1), full(N, 1), full(f3, 1), full(1, 1)],
        out_specs=[blk(_BB, N), stats_spec, stats_spec],
        out_shape=[jax.ShapeDtypeStruct((B, N), jnp.float32),
                   stats_shape, stats_shape],
        compiler_params=params,
    )(z3, adj, sc3.reshape(N, 1), sh3.reshape(N, 1), W4, b4.reshape(1, 1))
    sc4, sh4 = _finalize(ps, pq, B, g4, be4)

    out = pl.pallas_call(
        _head_kernel,
        out_shape=jax.ShapeDtypeStruct((B, 1), jnp.float32),
    )(z4, sc4.reshape(1, N), sh4.reshape(1, N), W5.reshape(1, N),
      b5.reshape(1, 1))
    return out


# speedup vs baseline: 1.6192x; 1.1958x over previous
import jax
import jax.numpy as jnp
from jax.experimental import pallas as pl
from jax.experimental.pallas import tpu as pltpu

_EPS = 1e-5
_BB = 128


def _lrelu(h):
    return jnp.where(h >= 0, h, 0.2 * h)


def _bdot(a, s):
    return jax.lax.dot_general(
        a, s, (((2,), (1,)), ((0,), (0,))), preferred_element_type=jnp.float32
    )


def _first_kernel(x_ref, c_ref, adj_ref, wx_ref, wc_ref, b_ref,
                  z_ref, ps_ref, pq_ref):
    bb, n, fx = x_ref.shape
    fc = c_ref.shape[-1]
    fo = wx_ref.shape[-1]
    s = jnp.dot(x_ref[...].reshape(bb * n, fx), wx_ref[...],
                preferred_element_type=jnp.float32)
    s = s + jnp.dot(c_ref[...].reshape(bb * n, fc), wc_ref[...],
                    preferred_element_type=jnp.float32)
    z = _bdot(adj_ref[...].astype(jnp.float32),
              s.reshape(bb, n, fo)) + b_ref[...]
    z_ref[...] = z.astype(z_ref.dtype)
    ps_ref[...] = jnp.sum(z, axis=(0, 2)).reshape(1, 1, n)
    pq_ref[...] = jnp.sum(z * z, axis=(0, 2)).reshape(1, 1, n)


def _mid_kernel(zp_ref, adj_ref, sc_ref, sh_ref, w_ref, b_ref,
                z_ref, ps_ref, pq_ref):
    bb, n, fi = zp_ref.shape
    fo = w_ref.shape[-1]
    h = _lrelu(zp_ref[...].astype(jnp.float32) * sc_ref[...][None]
               + sh_ref[...][None])
    s = jnp.dot(h.reshape(bb * n, fi), w_ref[...],
                preferred_element_type=jnp.float32)
    z = _bdot(adj_ref[...].astype(jnp.float32),
              s.reshape(bb, n, fo)) + b_ref[...]
    z_ref[...] = z.astype(z_ref.dtype)
    ps_ref[...] = jnp.sum(z, axis=(0, 2)).reshape(1, 1, n)
    pq_ref[...] = jnp.sum(z * z, axis=(0, 2)).reshape(1, 1, n)


def _last_gcn_kernel(zp_ref, adj_ref, sc_ref, sh_ref, w_ref, b_ref,
                     z_ref, ps_ref, pq_ref):
    bb, n, fi = zp_ref.shape
    h = _lrelu(zp_ref[...].astype(jnp.float32) * sc_ref[...][None]
               + sh_ref[...][None])
    s = jnp.sum(h * w_ref[...].reshape(1, 1, fi), axis=2)
    z = jnp.sum(adj_ref[...].astype(jnp.float32) * s[:, None, :],
                axis=2) + b_ref[...]
    z_ref[...] = z
    ps_ref[...] = jnp.sum(z, axis=0).reshape(1, 1, n)
    pq_ref[...] = jnp.sum(z * z, axis=0).reshape(1, 1, n)


def _head_kernel(z_ref, sc_ref, sh_ref, w5_ref, b5_ref, o_ref):
    h = _lrelu(z_ref[...] * sc_ref[...] + sh_ref[...])
    o = jnp.sum(h * w5_ref[...], axis=1, keepdims=True) + b5_ref[...]
    o_ref[...] = jax.nn.sigmoid(o)


def _finalize(ps, pq, cnt, g, be):
    s = jnp.sum(ps, axis=0).reshape(-1)
    q = jnp.sum(pq, axis=0).reshape(-1)
    mean = s / cnt
    var = q / cnt - mean * mean
    inv = jax.lax.rsqrt(var + _EPS)
    scale = g * inv
    shift = be - mean * scale
    return scale, shift


def kernel(x, adj, c, W1, b1, W2, b2, W3, b3, W4, b4,
           g1, be1, g2, be2, g3, be3, g4, be4, W5, b5):
    B, N, FX = x.shape
    FC = c.shape[-1]
    nblk = B // _BB
    grid = (nblk,)
    params = pltpu.CompilerParams(dimension_semantics=("parallel",))

    def blk(*shape):
        nd = len(shape)
        return pl.BlockSpec(shape, lambda i: (i,) + (0,) * (nd - 1))

    def full(*shape):
        nd = len(shape)
        return pl.BlockSpec(shape, lambda i: (0,) * nd)

    stats_shape = jax.ShapeDtypeStruct((nblk, 1, N), jnp.float32)
    stats_spec = pl.BlockSpec((1, 1, N), lambda i: (i, 0, 0))

    f1, f2, f3 = W1.shape[1], W2.shape[1], W3.shape[1]

    adjh = adj.astype(jnp.bfloat16)

    z1, ps, pq = pl.pallas_call(
        _first_kernel,
        grid=grid,
        in_specs=[blk(_BB, N, FX), blk(_BB, N, FC), blk(_BB, N, N),
                  full(FX, f1), full(FC, f1), full(1, f1)],
        out_specs=[blk(_BB, N, f1), stats_spec, stats_spec],
        out_shape=[jax.ShapeDtypeStruct((B, N, f1), jnp.bfloat16),
                   stats_shape, stats_shape],
        compiler_params=params,
    )(x, c, adjh, W1[:FX], W1[FX:], b1.reshape(1, f1))
    sc1, sh1 = _finalize(ps, pq, B * f1, g1, be1)

    def mid_pass(zp, fi, fo, w, b, sc, sh):
        return pl.pallas_call(
            _mid_kernel,
            grid=grid,
            in_specs=[blk(_BB, N, fi), blk(_BB, N, N),
                      full(N, 1), full(N, 1), full(fi, fo), full(1, fo)],
            out_specs=[blk(_BB, N, fo), stats_spec, stats_spec],
            out_shape=[jax.ShapeDtypeStruct((B, N, fo), jnp.bfloat16),
                       stats_shape, stats_shape],
            compiler_params=params,
        )(zp, adjh, sc.reshape(N, 1), sh.reshape(N, 1), w, b.reshape(1, fo))

    z2, ps, pq = mid_pass(z1, f1, f2, W2, b2, sc1, sh1)
    sc2, sh2 = _finalize(ps, pq, B * f2, g2, be2)

    z3, ps, pq = mid_pass(z2, f2, f3, W3, b3, sc2, sh2)
    sc3, sh3 = _finalize(ps, pq, B * f3, g3, be3)

    z4, ps, pq = pl.pallas_call(
        _last_gcn_kernel,
        grid=grid,
        in_specs=[blk(_BB, N, f3), blk(_BB, N, N),
                  full(N, 1), full(N, 1), full(f3, 1), full(1, 1)],
        out_specs=[blk(_BB, N), stats_spec, stats_spec],
        out_shape=[jax.ShapeDtypeStruct((B, N), jnp.float32),
                   stats_shape, stats_shape],
        compiler_params=params,
    )(z3, adjh, sc3.reshape(N, 1), sh3.reshape(N, 1), W4, b4.reshape(1, 1))
    sc4, sh4 = _finalize(ps, pq, B, g4, be4)

    out = pl.pallas_call(
        _head_kernel,
        out_shape=jax.ShapeDtypeStruct((B, 1), jnp.float32),
    )(z4, sc4.reshape(1, N), sh4.reshape(1, N), W5.reshape(1, N),
      b5.reshape(1, 1))
    return out
